# baseline (speedup 1.0000x reference)
import jax
import jax.numpy as jnp
from jax import lax
from jax.experimental import pallas as pl
from jax.experimental.pallas import tpu as pltpu
from jax.experimental.pallas import tpu_sc as plsc

NUM_BILLS = 1000000
NUM_LEGS = 100000
BILL_STRIDE = 1007616
LEG_STRIDE = 102400
BILL_BLK = BILL_STRIDE // 3
LEG_BLK = LEG_STRIDE // 1

BATCH = 16384
LATENT_DIM = 16
ROWS = LATENT_DIM + 1
NUM_WORKERS = 32
PAIRS_PER_WORKER = BATCH // NUM_WORKERS
CHUNK = 128
CHUNKS_PER_WORKER = PAIRS_PER_WORKER // CHUNK
GROUPS = PAIRS_PER_WORKER // 16


def _flatten_body(t_ref, b_ref, out_ref):
    d = pl.program_id(1)

    @pl.when(d < LATENT_DIM)
    def _():
        out_ref[...] = t_ref[d, :]

    @pl.when(d == LATENT_DIM)
    def _():
        out_ref[...] = b_ref[0, :]


def _dim_major_flat(table_t, bias_t, blk, blocks_per_row, stride):
    return pl.pallas_call(
        _flatten_body,
        grid=(blocks_per_row, ROWS),
        compiler_params=pltpu.CompilerParams(
            vmem_limit_bytes=120 * 1024 * 1024),
        in_specs=[
            pl.BlockSpec((LATENT_DIM, blk), lambda j, d: (0, j)),
            pl.BlockSpec((1, blk), lambda j, d: (0, j)),
        ],
        out_specs=pl.BlockSpec(
            (blk,), lambda j, d: (d * blocks_per_row + j,)),
        out_shape=jax.ShapeDtypeStruct((ROWS * stride,), jnp.float32),
    )(table_t, bias_t)


def _sc_body(bids, lids, gb, leg_t, bill_t, out_hbm,
             bidx, lidx, bgidx, lgidx, bcols, lcols, bb_v, lb_v, gb_v,
             out_v, sem, sem2, sem3, sem4, sem5):
    wid = lax.axis_index("s") * 2 + lax.axis_index("c")
    base = wid * PAIRS_PER_WORKER

    pltpu.sync_copy(bids.at[pl.ds(base, PAIRS_PER_WORKER)], bidx)
    pltpu.sync_copy(lids.at[pl.ds(base, PAIRS_PER_WORKER)], lidx)
    pltpu.sync_copy(gb, gb_v)

    def build(v, _):
        sl = pl.ds(v * 16, 16)
        bv = bidx[sl]
        lv = lidx[sl]
        for d in range(ROWS):
            bgidx[d, sl] = bv + d * BILL_STRIDE
            lgidx[d, sl] = lv + d * LEG_STRIDE
        return 0

    lax.fori_loop(0, GROUPS, build, 0, unroll=False)

    bias_copies = []
    for c in range(CHUNKS_PER_WORKER):
        sl = pl.ds(c * CHUNK, CHUNK)
        bias_copies.append(pltpu.async_copy(
            bill_t.at[bgidx.at[LATENT_DIM, sl]], bb_v.at[sl], sem3))
        bias_copies.append(pltpu.async_copy(
            leg_t.at[lgidx.at[LATENT_DIM, sl]], lb_v.at[sl], sem3))

    def fire(d, s):
        copies = []
        for c in range(CHUNKS_PER_WORKER):
            sl = pl.ds(c * CHUNK, CHUNK)
            copies.append(pltpu.async_copy(
                bill_t.at[bgidx.at[d, sl]], bcols.at[d, sl], s))
            copies.append(pltpu.async_copy(
                leg_t.at[lgidx.at[d, sl]], lcols.at[d, sl], s))
        return copies

    def gather_quad(i, _):
        d0 = i * 4
        cs = []
        for q, s in enumerate((sem, sem2, sem4, sem5)):
            cs.extend(fire(d0 + q, s))
        for cp in cs:
            cp.wait()
        return 0

    lax.fori_loop(0, LATENT_DIM // 4, gather_quad, 0, unroll=False)

    for cp in bias_copies:
        cp.wait()

    gbv = gb_v[...]

    def group(g, _):
        sl = pl.ds(g * 16, 16)
        acc = bcols[0, sl] * lcols[0, sl]
        for d in range(1, LATENT_DIM):
            acc = acc + bcols[d, sl] * lcols[d, sl]
        x = gbv + bb_v[sl] + lb_v[sl] + acc
        out_v[sl] = 1.0 / (1.0 + jnp.exp(-x))
        return 0

    lax.fori_loop(0, GROUPS, group, 0, unroll=False)

    pltpu.sync_copy(out_v, out_hbm.at[pl.ds(base, PAIRS_PER_WORKER)])


@jax.jit
def _predict(bids, lids, gb, leg_bias_t, bill_bias_t, leg_emb_t, bill_emb_t):
    leg_t = _dim_major_flat(leg_emb_t, leg_bias_t, LEG_BLK, 1, LEG_STRIDE)
    bill_t = _dim_major_flat(bill_emb_t, bill_bias_t, BILL_BLK, 3,
                             BILL_STRIDE)

    mesh = plsc.VectorSubcoreMesh(core_axis_name="c", subcore_axis_name="s")
    k = pl.kernel(
        _sc_body,
        out_type=jax.ShapeDtypeStruct((BATCH,), jnp.float32),
        mesh=mesh,
        compiler_params=pltpu.CompilerParams(needs_layout_passes=False,
                                             use_tc_tiling_on_sc=False),
        scratch_types=[
            pltpu.VMEM((PAIRS_PER_WORKER,), jnp.int32),
            pltpu.VMEM((PAIRS_PER_WORKER,), jnp.int32),
            pltpu.VMEM((ROWS, PAIRS_PER_WORKER), jnp.int32),
            pltpu.VMEM((ROWS, PAIRS_PER_WORKER), jnp.int32),
            pltpu.VMEM((LATENT_DIM, PAIRS_PER_WORKER), jnp.float32),
            pltpu.VMEM((LATENT_DIM, PAIRS_PER_WORKER), jnp.float32),
            pltpu.VMEM((PAIRS_PER_WORKER,), jnp.float32),
            pltpu.VMEM((PAIRS_PER_WORKER,), jnp.float32),
            pltpu.VMEM((16,), jnp.float32),
            pltpu.VMEM((PAIRS_PER_WORKER,), jnp.float32),
            pltpu.SemaphoreType.DMA,
            pltpu.SemaphoreType.DMA,
            pltpu.SemaphoreType.DMA,
            pltpu.SemaphoreType.DMA,
            pltpu.SemaphoreType.DMA,
        ],
    )
    return k(bids, lids, gb, leg_t, bill_t)


def kernel(bill_ids, legislator_ids, global_bias, legislator_bias, bill_bias,
           legislator_embedding, bill_embedding):
    bids = bill_ids.astype(jnp.int32)
    lids = legislator_ids.astype(jnp.int32)
    gb = jnp.broadcast_to(jnp.reshape(global_bias, (1,)), (16,))
    leg_bias_t = jnp.transpose(legislator_bias)
    bill_bias_t = jnp.transpose(bill_bias)
    leg_emb_t = jnp.transpose(legislator_embedding)
    bill_emb_t = jnp.transpose(bill_embedding)
    return _predict(bids, lids, gb, leg_bias_t, bill_bias_t,
                    leg_emb_t, bill_emb_t)


# speedup vs baseline: 1.0014x; 1.0014x over previous
import jax
import jax.numpy as jnp
from jax import lax
from jax.experimental import pallas as pl
from jax.experimental.pallas import tpu as pltpu
from jax.experimental.pallas import tpu_sc as plsc

NUM_BILLS = 1000000
NUM_LEGS = 100000
BILL_STRIDE = 1007616
LEG_STRIDE = 102400
BILL_BLK = BILL_STRIDE // 3
LEG_BLK = LEG_STRIDE // 1

BATCH = 16384
LATENT_DIM = 16
ROWS = LATENT_DIM + 1
NUM_WORKERS = 32
PAIRS_PER_WORKER = BATCH // NUM_WORKERS
CHUNK = 128
CHUNKS_PER_WORKER = PAIRS_PER_WORKER // CHUNK
GROUPS = PAIRS_PER_WORKER // 16


def _flatten_body(bt_ref, bb_ref, lt_ref, lb_ref, bill_out, leg_out):
    d = pl.program_id(1)

    @pl.when(d < LATENT_DIM)
    def _():
        bill_out[...] = bt_ref[d, :]
        leg_out[...] = lt_ref[d, :]

    @pl.when(d == LATENT_DIM)
    def _():
        bill_out[...] = bb_ref[0, :]
        leg_out[...] = lb_ref[0, :]


def _dim_major_flats(bill_t, bill_b, leg_t, leg_b):
    return pl.pallas_call(
        _flatten_body,
        grid=(3, ROWS),
        in_specs=[
            pl.BlockSpec((LATENT_DIM, BILL_BLK), lambda j, d: (0, j)),
            pl.BlockSpec((1, BILL_BLK), lambda j, d: (0, j)),
            pl.BlockSpec((LATENT_DIM, LEG_BLK), lambda j, d: (0, 0)),
            pl.BlockSpec((1, LEG_BLK), lambda j, d: (0, 0)),
        ],
        out_specs=[
            pl.BlockSpec((BILL_BLK,), lambda j, d: (d * 3 + j,)),
            pl.BlockSpec((LEG_BLK,), lambda j, d: (d,)),
        ],
        out_shape=[
            jax.ShapeDtypeStruct((ROWS * BILL_STRIDE,), jnp.float32),
            jax.ShapeDtypeStruct((ROWS * LEG_STRIDE,), jnp.float32),
        ],
    )(bill_t, bill_b, leg_t, leg_b)


def _sc_body(bids, lids, gb, leg_t, bill_t, out_hbm,
             bidx, lidx, bgidx, lgidx, bcols, lcols, bb_v, lb_v, gb_v,
             out_v, sem, sem2, sem3, sem4, sem5):
    wid = lax.axis_index("s") * 2 + lax.axis_index("c")
    base = wid * PAIRS_PER_WORKER

    pltpu.sync_copy(bids.at[pl.ds(base, PAIRS_PER_WORKER)], bidx)
    pltpu.sync_copy(lids.at[pl.ds(base, PAIRS_PER_WORKER)], lidx)
    pltpu.sync_copy(gb, gb_v)

    def build(v, _):
        sl = pl.ds(v * 16, 16)
        bv = bidx[sl]
        lv = lidx[sl]
        for d in range(ROWS):
            bgidx[d, sl] = bv + d * BILL_STRIDE
            lgidx[d, sl] = lv + d * LEG_STRIDE
        return 0

    lax.fori_loop(0, GROUPS, build, 0, unroll=False)

    bias_copies = []
    for c in range(CHUNKS_PER_WORKER):
        sl = pl.ds(c * CHUNK, CHUNK)
        bias_copies.append(pltpu.async_copy(
            bill_t.at[bgidx.at[LATENT_DIM, sl]], bb_v.at[sl], sem3))
        bias_copies.append(pltpu.async_copy(
            leg_t.at[lgidx.at[LATENT_DIM, sl]], lb_v.at[sl], sem3))

    def fire(d, s):
        copies = []
        for c in range(CHUNKS_PER_WORKER):
            sl = pl.ds(c * CHUNK, CHUNK)
            copies.append(pltpu.async_copy(
                bill_t.at[bgidx.at[d, sl]], bcols.at[d, sl], s))
            copies.append(pltpu.async_copy(
                leg_t.at[lgidx.at[d, sl]], lcols.at[d, sl], s))
        return copies

    def gather_quad(i, _):
        d0 = i * 4
        cs = []
        for q, s in enumerate((sem, sem2, sem4, sem5)):
            cs.extend(fire(d0 + q, s))
        for cp in cs:
            cp.wait()
        return 0

    lax.fori_loop(0, LATENT_DIM // 4, gather_quad, 0, unroll=False)

    for cp in bias_copies:
        cp.wait()

    gbv = gb_v[...]

    def group(g, _):
        sl = pl.ds(g * 16, 16)
        acc = bcols[0, sl] * lcols[0, sl]
        for d in range(1, LATENT_DIM):
            acc = acc + bcols[d, sl] * lcols[d, sl]
        x = gbv + bb_v[sl] + lb_v[sl] + acc
        out_v[sl] = 1.0 / (1.0 + jnp.exp(-x))
        return 0

    lax.fori_loop(0, GROUPS, group, 0, unroll=False)

    pltpu.sync_copy(out_v, out_hbm.at[pl.ds(base, PAIRS_PER_WORKER)])


@jax.jit
def _predict(bids, lids, gb, leg_bias_t, bill_bias_t, leg_emb_t, bill_emb_t):
    bill_t, leg_t = _dim_major_flats(bill_emb_t, bill_bias_t,
                                     leg_emb_t, leg_bias_t)

    mesh = plsc.VectorSubcoreMesh(core_axis_name="c", subcore_axis_name="s")
    k = pl.kernel(
        _sc_body,
        out_type=jax.ShapeDtypeStruct((BATCH,), jnp.float32),
        mesh=mesh,
        compiler_params=pltpu.CompilerParams(needs_layout_passes=False,
                                             use_tc_tiling_on_sc=False),
        scratch_types=[
            pltpu.VMEM((PAIRS_PER_WORKER,), jnp.int32),
            pltpu.VMEM((PAIRS_PER_WORKER,), jnp.int32),
            pltpu.VMEM((ROWS, PAIRS_PER_WORKER), jnp.int32),
            pltpu.VMEM((ROWS, PAIRS_PER_WORKER), jnp.int32),
            pltpu.VMEM((LATENT_DIM, PAIRS_PER_WORKER), jnp.float32),
            pltpu.VMEM((LATENT_DIM, PAIRS_PER_WORKER), jnp.float32),
            pltpu.VMEM((PAIRS_PER_WORKER,), jnp.float32),
            pltpu.VMEM((PAIRS_PER_WORKER,), jnp.float32),
            pltpu.VMEM((16,), jnp.float32),
            pltpu.VMEM((PAIRS_PER_WORKER,), jnp.float32),
            pltpu.SemaphoreType.DMA,
            pltpu.SemaphoreType.DMA,
            pltpu.SemaphoreType.DMA,
            pltpu.SemaphoreType.DMA,
            pltpu.SemaphoreType.DMA,
        ],
    )
    return k(bids, lids, gb, leg_t, bill_t)


def kernel(bill_ids, legislator_ids, global_bias, legislator_bias, bill_bias,
           legislator_embedding, bill_embedding):
    bids = bill_ids.astype(jnp.int32)
    lids = legislator_ids.astype(jnp.int32)
    gb = jnp.broadcast_to(jnp.reshape(global_bias, (1,)), (16,))
    leg_bias_t = jnp.transpose(legislator_bias)
    bill_bias_t = jnp.transpose(bill_bias)
    leg_emb_t = jnp.transpose(legislator_embedding)
    bill_emb_t = jnp.transpose(bill_embedding)
    return _predict(bids, lids, gb, leg_bias_t, bill_bias_t,
                    leg_emb_t, bill_emb_t)


# speedup vs baseline: 1.1931x; 1.1914x over previous
import jax
import jax.numpy as jnp
from jax import lax
from jax.experimental import pallas as pl
from jax.experimental.pallas import tpu as pltpu
from jax.experimental.pallas import tpu_sc as plsc

NUM_BILLS = 1000000
NUM_LEGS = 100000
BILL_STRIDE = 1007616
LEG_STRIDE = 102400
BILL_BLK = BILL_STRIDE // 3
LEG_BLK = LEG_STRIDE // 1

BATCH = 16384
LATENT_DIM = 16
ROWS = LATENT_DIM + 1
NUM_WORKERS = 32
PAIRS_PER_WORKER = BATCH // NUM_WORKERS
CHUNK = 128
CHUNKS_PER_WORKER = PAIRS_PER_WORKER // CHUNK
GROUPS = PAIRS_PER_WORKER // 16


def _flatten_body(t_ref, b_ref, out_ref):
    d = pl.program_id(1)

    @pl.when(d < LATENT_DIM)
    def _():
        out_ref[...] = t_ref[d, :]

    @pl.when(d == LATENT_DIM)
    def _():
        out_ref[...] = b_ref[0, :]


BCH = 131072
BCHUNKS = 8


def _split_body(t_ref, b_ref, *out_refs):
    for d in range(LATENT_DIM):
        out_refs[d][...] = t_ref[d, :]
    out_refs[LATENT_DIM][...] = b_ref[0, :]


def _bill_columns(table_t, bias_t):
    return pl.pallas_call(
        _split_body,
        grid=(BCHUNKS,),
        in_specs=[
            pl.BlockSpec((LATENT_DIM, BCH), lambda j: (0, j)),
            pl.BlockSpec((1, BCH), lambda j: (0, j)),
        ],
        out_specs=[pl.BlockSpec((BCH,), lambda j: (j,))
                   for _ in range(ROWS)],
        out_shape=[jax.ShapeDtypeStruct((BCHUNKS * BCH,), jnp.float32)
                   for _ in range(ROWS)],
    )(table_t, bias_t)


def _dim_major_flat(table_t, bias_t, blk, blocks_per_row, stride):
    return pl.pallas_call(
        _flatten_body,
        grid=(blocks_per_row, ROWS),
        compiler_params=pltpu.CompilerParams(
            vmem_limit_bytes=120 * 1024 * 1024),
        in_specs=[
            pl.BlockSpec((LATENT_DIM, blk), lambda j, d: (0, j)),
            pl.BlockSpec((1, blk), lambda j, d: (0, j)),
        ],
        out_specs=pl.BlockSpec(
            (blk,), lambda j, d: (d * blocks_per_row + j,)),
        out_shape=jax.ShapeDtypeStruct((ROWS * stride,), jnp.float32),
    )(table_t, bias_t)


def _sc_body(bids, lids, gb, leg_t, *rest):
    (bt0, bt1, bt2, bt3, bt4, bt5, bt6, bt7, bt8, bt9, bt10, bt11, bt12,
     bt13, bt14, bt15, bt16, out_hbm,
     bidx, lidx, lgidx, bcols, lcols, bb_v, lb_v, gb_v,
     out_v, sem, sem2, sem3, sem4) = rest
    bts = (bt0, bt1, bt2, bt3, bt4, bt5, bt6, bt7, bt8, bt9, bt10, bt11,
           bt12, bt13, bt14, bt15, bt16)
    sems = (sem, sem2, sem3, sem4)
    wid = lax.axis_index("s") * 2 + lax.axis_index("c")
    base = wid * PAIRS_PER_WORKER

    pltpu.sync_copy(bids.at[pl.ds(base, PAIRS_PER_WORKER)], bidx)
    pltpu.sync_copy(lids.at[pl.ds(base, PAIRS_PER_WORKER)], lidx)
    pltpu.sync_copy(gb, gb_v)

    def build(v, _):
        sl = pl.ds(v * 16, 16)
        lv = lidx[sl]
        for d in range(ROWS):
            lgidx[d, sl] = lv + d * LEG_STRIDE
        return 0

    lax.fori_loop(0, GROUPS, build, 0, unroll=False)

    def fire(d, s):
        copies = []
        for c in range(CHUNKS_PER_WORKER):
            sl = pl.ds(c * CHUNK, CHUNK)
            bdst = bb_v.at[sl] if d == LATENT_DIM else bcols.at[d, sl]
            ldst = lb_v.at[sl] if d == LATENT_DIM else lcols.at[d, sl]
            copies.append(pltpu.async_copy(bts[d].at[bidx.at[sl]], bdst, s))
            copies.append(pltpu.async_copy(
                leg_t.at[lgidx.at[d, sl]], ldst, s))
        return copies

    inflight = {}
    for d in range(ROWS):
        inflight[d] = fire(d, sems[d % 4])
        if d >= 4:
            for cp in inflight.pop(d - 4):
                cp.wait()
    for d in sorted(inflight):
        for cp in inflight[d]:
            cp.wait()

    gbv = gb_v[...]

    def group(g, _):
        sl = pl.ds(g * 16, 16)
        acc = bcols[0, sl] * lcols[0, sl]
        for d in range(1, LATENT_DIM):
            acc = acc + bcols[d, sl] * lcols[d, sl]
        x = gbv + bb_v[sl] + lb_v[sl] + acc
        out_v[sl] = 1.0 / (1.0 + jnp.exp(-x))
        return 0

    lax.fori_loop(0, GROUPS, group, 0, unroll=False)

    pltpu.sync_copy(out_v, out_hbm.at[pl.ds(base, PAIRS_PER_WORKER)])


@jax.jit
def _predict(bids, lids, gb, leg_bias_t, bill_bias_t, leg_emb_t, bill_emb_t):
    leg_t = _dim_major_flat(leg_emb_t, leg_bias_t, LEG_BLK, 1, LEG_STRIDE)
    bill_cols = _bill_columns(bill_emb_t, bill_bias_t)

    mesh = plsc.VectorSubcoreMesh(core_axis_name="c", subcore_axis_name="s")
    k = pl.kernel(
        _sc_body,
        out_type=jax.ShapeDtypeStruct((BATCH,), jnp.float32),
        mesh=mesh,
        compiler_params=pltpu.CompilerParams(needs_layout_passes=False,
                                             use_tc_tiling_on_sc=False),
        scratch_types=[
            pltpu.VMEM((PAIRS_PER_WORKER,), jnp.int32),
            pltpu.VMEM((PAIRS_PER_WORKER,), jnp.int32),
            pltpu.VMEM((ROWS, PAIRS_PER_WORKER), jnp.int32),
            pltpu.VMEM((LATENT_DIM, PAIRS_PER_WORKER), jnp.float32),
            pltpu.VMEM((LATENT_DIM, PAIRS_PER_WORKER), jnp.float32),
            pltpu.VMEM((PAIRS_PER_WORKER,), jnp.float32),
            pltpu.VMEM((PAIRS_PER_WORKER,), jnp.float32),
            pltpu.VMEM((16,), jnp.float32),
            pltpu.VMEM((PAIRS_PER_WORKER,), jnp.float32),
            pltpu.SemaphoreType.DMA,
            pltpu.SemaphoreType.DMA,
            pltpu.SemaphoreType.DMA,
            pltpu.SemaphoreType.DMA,
        ],
    )
    return k(bids, lids, gb, leg_t, *bill_cols)


def kernel(bill_ids, legislator_ids, global_bias, legislator_bias, bill_bias,
           legislator_embedding, bill_embedding):
    bids = bill_ids.astype(jnp.int32)
    lids = legislator_ids.astype(jnp.int32)
    gb = jnp.broadcast_to(jnp.reshape(global_bias, (1,)), (16,))
    leg_bias_t = jnp.transpose(legislator_bias)
    bill_bias_t = jnp.transpose(bill_bias)
    leg_emb_t = jnp.transpose(legislator_embedding)
    bill_emb_t = jnp.transpose(bill_embedding)
    return _predict(bids, lids, gb, leg_bias_t, bill_bias_t,
                    leg_emb_t, bill_emb_t)


# speedup vs baseline: 1.2518x; 1.0492x over previous
import jax
import jax.numpy as jnp
from jax import lax
from jax.experimental import pallas as pl
from jax.experimental.pallas import tpu as pltpu
from jax.experimental.pallas import tpu_sc as plsc

NUM_BILLS = 1000000
NUM_LEGS = 100000
BILL_STRIDE = 1007616
LEG_STRIDE = 102400
BILL_BLK = BILL_STRIDE // 3
LEG_BLK = LEG_STRIDE // 1

BATCH = 16384
LATENT_DIM = 16
ROWS = LATENT_DIM + 1
NUM_WORKERS = 32
PAIRS_PER_WORKER = BATCH // NUM_WORKERS
CHUNK = 128
CHUNKS_PER_WORKER = PAIRS_PER_WORKER // CHUNK
GROUPS = PAIRS_PER_WORKER // 16


def _flatten_body(t_ref, b_ref, out_ref):
    d = pl.program_id(1)

    @pl.when(d < LATENT_DIM)
    def _():
        out_ref[...] = t_ref[d, :]

    @pl.when(d == LATENT_DIM)
    def _():
        out_ref[...] = b_ref[0, :]


BCH = 131072
BCHUNKS = 8
LCH = 102400


def _split_body(t_ref, b_ref, *out_refs):
    for d in range(LATENT_DIM):
        out_refs[d][...] = t_ref[d, :]
    out_refs[LATENT_DIM][...] = b_ref[0, :]


def _columns(table_t, bias_t, ch, chunks):
    return pl.pallas_call(
        _split_body,
        grid=(chunks,),
        in_specs=[
            pl.BlockSpec((LATENT_DIM, ch), lambda j: (0, j)),
            pl.BlockSpec((1, ch), lambda j: (0, j)),
        ],
        out_specs=[pl.BlockSpec((ch,), lambda j: (j,))
                   for _ in range(ROWS)],
        out_shape=[jax.ShapeDtypeStruct((chunks * ch,), jnp.float32)
                   for _ in range(ROWS)],
    )(table_t, bias_t)


def _dim_major_flat(table_t, bias_t, blk, blocks_per_row, stride):
    return pl.pallas_call(
        _flatten_body,
        grid=(blocks_per_row, ROWS),
        compiler_params=pltpu.CompilerParams(
            vmem_limit_bytes=120 * 1024 * 1024),
        in_specs=[
            pl.BlockSpec((LATENT_DIM, blk), lambda j, d: (0, j)),
            pl.BlockSpec((1, blk), lambda j, d: (0, j)),
        ],
        out_specs=pl.BlockSpec(
            (blk,), lambda j, d: (d * blocks_per_row + j,)),
        out_shape=jax.ShapeDtypeStruct((ROWS * stride,), jnp.float32),
    )(table_t, bias_t)


def _sc_body(bids, lids, gb, *rest):
    (lt0, lt1, lt2, lt3, lt4, lt5, lt6, lt7, lt8, lt9, lt10, lt11, lt12,
     lt13, lt14, lt15, lt16,
     bt0, bt1, bt2, bt3, bt4, bt5, bt6, bt7, bt8, bt9, bt10, bt11, bt12,
     bt13, bt14, bt15, bt16, out_hbm,
     bidx, lidx, bcols, lcols, bb_v, lb_v, gb_v,
     out_v, sem, sem2, sem3, sem4) = rest
    lts = (lt0, lt1, lt2, lt3, lt4, lt5, lt6, lt7, lt8, lt9, lt10, lt11,
           lt12, lt13, lt14, lt15, lt16)
    bts = (bt0, bt1, bt2, bt3, bt4, bt5, bt6, bt7, bt8, bt9, bt10, bt11,
           bt12, bt13, bt14, bt15, bt16)
    sems = (sem, sem2, sem3, sem4)
    wid = lax.axis_index("s") * 2 + lax.axis_index("c")
    base = wid * PAIRS_PER_WORKER

    pltpu.sync_copy(bids.at[pl.ds(base, PAIRS_PER_WORKER)], bidx)
    pltpu.sync_copy(lids.at[pl.ds(base, PAIRS_PER_WORKER)], lidx)
    pltpu.sync_copy(gb, gb_v)

    def fire(d, s):
        copies = []
        for c in range(CHUNKS_PER_WORKER):
            sl = pl.ds(c * CHUNK, CHUNK)
            bdst = bb_v.at[sl] if d == LATENT_DIM else bcols.at[d, sl]
            ldst = lb_v.at[sl] if d == LATENT_DIM else lcols.at[d, sl]
            copies.append(pltpu.async_copy(bts[d].at[bidx.at[sl]], bdst, s))
            copies.append(pltpu.async_copy(lts[d].at[lidx.at[sl]], ldst, s))
        return copies

    inflight = {}
    for d in range(ROWS):
        inflight[d] = fire(d, sems[d % 4])
        if d >= 4:
            for cp in inflight.pop(d - 4):
                cp.wait()
    for d in sorted(inflight):
        for cp in inflight[d]:
            cp.wait()

    gbv = gb_v[...]

    def group(g, _):
        sl = pl.ds(g * 16, 16)
        acc = bcols[0, sl] * lcols[0, sl]
        for d in range(1, LATENT_DIM):
            acc = acc + bcols[d, sl] * lcols[d, sl]
        x = gbv + bb_v[sl] + lb_v[sl] + acc
        out_v[sl] = 1.0 / (1.0 + jnp.exp(-x))
        return 0

    lax.fori_loop(0, GROUPS, group, 0, unroll=False)

    pltpu.sync_copy(out_v, out_hbm.at[pl.ds(base, PAIRS_PER_WORKER)])


@jax.jit
def _predict(bids, lids, gb, leg_bias_t, bill_bias_t, leg_emb_t, bill_emb_t):
    leg_cols = _columns(leg_emb_t, leg_bias_t, LCH, 1)
    bill_cols = _columns(bill_emb_t, bill_bias_t, BCH, BCHUNKS)

    mesh = plsc.VectorSubcoreMesh(core_axis_name="c", subcore_axis_name="s")
    k = pl.kernel(
        _sc_body,
        out_type=jax.ShapeDtypeStruct((BATCH,), jnp.float32),
        mesh=mesh,
        compiler_params=pltpu.CompilerParams(needs_layout_passes=False,
                                             use_tc_tiling_on_sc=False),
        scratch_types=[
            pltpu.VMEM((PAIRS_PER_WORKER,), jnp.int32),
            pltpu.VMEM((PAIRS_PER_WORKER,), jnp.int32),
            pltpu.VMEM((LATENT_DIM, PAIRS_PER_WORKER), jnp.float32),
            pltpu.VMEM((LATENT_DIM, PAIRS_PER_WORKER), jnp.float32),
            pltpu.VMEM((PAIRS_PER_WORKER,), jnp.float32),
            pltpu.VMEM((PAIRS_PER_WORKER,), jnp.float32),
            pltpu.VMEM((16,), jnp.float32),
            pltpu.VMEM((PAIRS_PER_WORKER,), jnp.float32),
            pltpu.SemaphoreType.DMA,
            pltpu.SemaphoreType.DMA,
            pltpu.SemaphoreType.DMA,
            pltpu.SemaphoreType.DMA,
        ],
    )
    return k(bids, lids, gb, *leg_cols, *bill_cols)


def kernel(bill_ids, legislator_ids, global_bias, legislator_bias, bill_bias,
           legislator_embedding, bill_embedding):
    bids = bill_ids.astype(jnp.int32)
    lids = legislator_ids.astype(jnp.int32)
    gb = jnp.broadcast_to(jnp.reshape(global_bias, (1,)), (16,))
    leg_bias_t = jnp.transpose(legislator_bias)
    bill_bias_t = jnp.transpose(bill_bias)
    leg_emb_t = jnp.transpose(legislator_embedding)
    bill_emb_t = jnp.transpose(bill_embedding)
    return _predict(bids, lids, gb, leg_bias_t, bill_bias_t,
                    leg_emb_t, bill_emb_t)
